# baseline (speedup 1.0000x reference)
import functools

import numpy as np
import jax
import jax.numpy as jnp
from jax.experimental import pallas as pl
from jax.experimental.pallas import tpu as pltpu

_N_BINS = 15
_BLOCK_R = 2048

_bounds = np.linspace(0.0, 1.0, _N_BINS + 1, dtype=np.float32)
_UP_COL = np.full((16, 1), 2.0, dtype=np.float32)
_UP_COL[:14, 0] = _bounds[1:15]
_UP_COL[14, 0] = -1.0


def _ece_block_kernel(x_ref, lab_ref, up_ref, out_ref,
                      cnt_ref, cf_ref, ac_ref, *, inv_n):
    i = pl.program_id(0)

    @pl.when(i == 0)
    def _init():
        cnt_ref[...] = jnp.zeros_like(cnt_ref)
        cf_ref[...] = jnp.zeros_like(cf_ref)
        ac_ref[...] = jnp.zeros_like(ac_ref)

    x = x_ref[...]
    c = x.shape[1]
    m = jnp.max(x, axis=1, keepdims=True)
    ez = jnp.exp(x - m)
    eqb = (x == m).astype(jnp.bfloat16)

    ones_row = jnp.ones((1, c), jnp.float32)
    iota_row = jax.lax.broadcasted_iota(jnp.int32, (1, c), 1).astype(jnp.bfloat16)
    dn = (((1,), (1,)), ((), ()))
    s = jax.lax.dot_general(ones_row, ez, dn,
                            precision=jax.lax.Precision.HIGHEST)
    pred = jax.lax.dot_general(iota_row, eqb, dn,
                               preferred_element_type=jnp.float32)
    conf = 1.0 / s
    lab = lab_ref[0].astype(jnp.float32)
    acc = (pred == lab).astype(jnp.float32)

    mask = (conf > up_ref[...]).astype(jnp.float32)
    cnt_ref[...] += mask
    cf_ref[...] += mask * conf
    ac_ref[...] += mask * acc

    @pl.when(i == pl.num_programs(0) - 1)
    def _fini():
        cum = jnp.concatenate(
            [jnp.sum(cnt_ref[...], axis=1, keepdims=True),
             jnp.sum(cf_ref[...], axis=1, keepdims=True),
             jnp.sum(ac_ref[...], axis=1, keepdims=True)], axis=1)
        total = cum[14:15, :]
        prev = jnp.concatenate([total, cum[0:14, :]], axis=0)
        cur = jnp.concatenate(
            [cum[0:14, :], jnp.zeros((1, 3), jnp.float32)], axis=0)
        stats = prev - cur
        cb = stats[:, 0:1]
        safe = jnp.maximum(cb, 1.0)
        contrib = jnp.abs(stats[:, 1:2] - stats[:, 2:3]) / safe * (cb * inv_n)
        contrib = jnp.where(cb > 0.0, contrib, 0.0)
        out_ref[...] = jnp.sum(contrib, axis=0, keepdims=True)


def kernel(logits_input, labels_input):
    n, c = logits_input.shape
    grid = n // _BLOCK_R
    labels = labels_input.astype(jnp.int32).reshape(grid, 1, _BLOCK_R)
    out = pl.pallas_call(
        functools.partial(_ece_block_kernel, inv_n=1.0 / n),
        grid=(grid,),
        in_specs=[
            pl.BlockSpec((_BLOCK_R, c), lambda i: (i, 0)),
            pl.BlockSpec((1, 1, _BLOCK_R), lambda i: (i, 0, 0)),
            pl.BlockSpec((16, 1), lambda i: (0, 0)),
        ],
        out_specs=pl.BlockSpec((1, 1), lambda i: (0, 0)),
        out_shape=jax.ShapeDtypeStruct((1, 1), jnp.float32),
        scratch_shapes=[pltpu.VMEM((16, _BLOCK_R), jnp.float32),
                        pltpu.VMEM((16, _BLOCK_R), jnp.float32),
                        pltpu.VMEM((16, _BLOCK_R), jnp.float32)],
    )(logits_input, labels, jnp.asarray(_UP_COL))
    return out.reshape(1)


# speedup vs baseline: 1.0962x; 1.0962x over previous
import functools

import numpy as np
import jax
import jax.numpy as jnp
from jax.experimental import pallas as pl
from jax.experimental.pallas import tpu as pltpu

_N_BINS = 15
_BLOCK_R = 8192

_bounds = np.linspace(0.0, 1.0, _N_BINS + 1, dtype=np.float32)
_UP_COL = np.full((16, 1), 2.0, dtype=np.float32)
_UP_COL[:14, 0] = _bounds[1:15]
_UP_COL[14, 0] = -1.0


def _ece_block_kernel(x_ref, lab_ref, up_ref, out_ref,
                      cnt_ref, cf_ref, ac_ref, *, inv_n):
    i = pl.program_id(0)

    @pl.when(i == 0)
    def _init():
        cnt_ref[...] = jnp.zeros_like(cnt_ref)
        cf_ref[...] = jnp.zeros_like(cf_ref)
        ac_ref[...] = jnp.zeros_like(ac_ref)

    x = x_ref[...]
    c = x.shape[1]
    m = jnp.max(x, axis=1, keepdims=True)
    ez = jnp.exp(x - m)
    eqb = (x == m).astype(jnp.bfloat16)

    ones_row = jnp.ones((1, c), jnp.float32)
    iota_row = jax.lax.broadcasted_iota(jnp.int32, (1, c), 1).astype(jnp.bfloat16)
    dn = (((1,), (1,)), ((), ()))
    s = jax.lax.dot_general(ones_row, ez, dn,
                            precision=jax.lax.Precision.HIGHEST)
    pred = jax.lax.dot_general(iota_row, eqb, dn,
                               preferred_element_type=jnp.float32)
    conf = 1.0 / s
    lab = lab_ref[0].astype(jnp.float32)
    acc = (pred == lab).astype(jnp.float32)

    mask = (conf > up_ref[...]).astype(jnp.float32)
    cnt_ref[...] += mask
    cf_ref[...] += mask * conf
    ac_ref[...] += mask * acc

    @pl.when(i == pl.num_programs(0) - 1)
    def _fini():
        cum = jnp.concatenate(
            [jnp.sum(cnt_ref[...], axis=1, keepdims=True),
             jnp.sum(cf_ref[...], axis=1, keepdims=True),
             jnp.sum(ac_ref[...], axis=1, keepdims=True)], axis=1)
        total = cum[14:15, :]
        prev = jnp.concatenate([total, cum[0:14, :]], axis=0)
        cur = jnp.concatenate(
            [cum[0:14, :], jnp.zeros((1, 3), jnp.float32)], axis=0)
        stats = prev - cur
        cb = stats[:, 0:1]
        safe = jnp.maximum(cb, 1.0)
        contrib = jnp.abs(stats[:, 1:2] - stats[:, 2:3]) / safe * (cb * inv_n)
        contrib = jnp.where(cb > 0.0, contrib, 0.0)
        out_ref[...] = jnp.sum(contrib, axis=0, keepdims=True)


def kernel(logits_input, labels_input):
    n, c = logits_input.shape
    grid = n // _BLOCK_R
    labels = labels_input.astype(jnp.int32).reshape(grid, 1, _BLOCK_R)
    out = pl.pallas_call(
        functools.partial(_ece_block_kernel, inv_n=1.0 / n),
        grid=(grid,),
        in_specs=[
            pl.BlockSpec((_BLOCK_R, c), lambda i: (i, 0)),
            pl.BlockSpec((1, 1, _BLOCK_R), lambda i: (i, 0, 0)),
            pl.BlockSpec((16, 1), lambda i: (0, 0)),
        ],
        out_specs=pl.BlockSpec((1, 1), lambda i: (0, 0)),
        out_shape=jax.ShapeDtypeStruct((1, 1), jnp.float32),
        scratch_shapes=[pltpu.VMEM((16, _BLOCK_R), jnp.float32),
                        pltpu.VMEM((16, _BLOCK_R), jnp.float32),
                        pltpu.VMEM((16, _BLOCK_R), jnp.float32)],
    )(logits_input, labels, jnp.asarray(_UP_COL))
    return out.reshape(1)


# speedup vs baseline: 1.1130x; 1.0153x over previous
import functools

import numpy as np
import jax
import jax.numpy as jnp
from jax.experimental import pallas as pl
from jax.experimental.pallas import tpu as pltpu

_N_BINS = 15
_BLOCK_R = 16384

_bounds = np.linspace(0.0, 1.0, _N_BINS + 1, dtype=np.float32)
_UP_COL = np.full((16, 1), 2.0, dtype=np.float32)
_UP_COL[:14, 0] = _bounds[1:15]
_UP_COL[14, 0] = -1.0


def _ece_block_kernel(x_ref, lab_ref, up_ref, out_ref,
                      cnt_ref, cf_ref, ac_ref, *, inv_n):
    i = pl.program_id(0)

    @pl.when(i == 0)
    def _init():
        cnt_ref[...] = jnp.zeros_like(cnt_ref)
        cf_ref[...] = jnp.zeros_like(cf_ref)
        ac_ref[...] = jnp.zeros_like(ac_ref)

    x = x_ref[...]
    c = x.shape[1]
    m = jnp.max(x, axis=1, keepdims=True)
    ez = jnp.exp(x - m)
    eqb = (x == m).astype(jnp.bfloat16)

    ones_row = jnp.ones((1, c), jnp.float32)
    iota_row = jax.lax.broadcasted_iota(jnp.int32, (1, c), 1).astype(jnp.bfloat16)
    dn = (((1,), (1,)), ((), ()))
    s = jax.lax.dot_general(ones_row, ez, dn,
                            precision=jax.lax.Precision.HIGHEST)
    pred = jax.lax.dot_general(iota_row, eqb, dn,
                               preferred_element_type=jnp.float32)
    conf = 1.0 / s
    lab = lab_ref[0].astype(jnp.float32)
    acc = (pred == lab).astype(jnp.float32)

    mask = (conf > up_ref[...]).astype(jnp.float32)
    cnt_ref[...] += mask
    cf_ref[...] += mask * conf
    ac_ref[...] += mask * acc

    @pl.when(i == pl.num_programs(0) - 1)
    def _fini():
        cum = jnp.concatenate(
            [jnp.sum(cnt_ref[...], axis=1, keepdims=True),
             jnp.sum(cf_ref[...], axis=1, keepdims=True),
             jnp.sum(ac_ref[...], axis=1, keepdims=True)], axis=1)
        total = cum[14:15, :]
        prev = jnp.concatenate([total, cum[0:14, :]], axis=0)
        cur = jnp.concatenate(
            [cum[0:14, :], jnp.zeros((1, 3), jnp.float32)], axis=0)
        stats = prev - cur
        cb = stats[:, 0:1]
        safe = jnp.maximum(cb, 1.0)
        contrib = jnp.abs(stats[:, 1:2] - stats[:, 2:3]) / safe * (cb * inv_n)
        contrib = jnp.where(cb > 0.0, contrib, 0.0)
        out_ref[...] = jnp.sum(contrib, axis=0, keepdims=True)


def kernel(logits_input, labels_input):
    n, c = logits_input.shape
    grid = n // _BLOCK_R
    labels = labels_input.astype(jnp.int32).reshape(grid, 1, _BLOCK_R)
    out = pl.pallas_call(
        functools.partial(_ece_block_kernel, inv_n=1.0 / n),
        grid=(grid,),
        in_specs=[
            pl.BlockSpec((_BLOCK_R, c), lambda i: (i, 0)),
            pl.BlockSpec((1, 1, _BLOCK_R), lambda i: (i, 0, 0)),
            pl.BlockSpec((16, 1), lambda i: (0, 0)),
        ],
        out_specs=pl.BlockSpec((1, 1), lambda i: (0, 0)),
        out_shape=jax.ShapeDtypeStruct((1, 1), jnp.float32),
        scratch_shapes=[pltpu.VMEM((16, _BLOCK_R), jnp.float32),
                        pltpu.VMEM((16, _BLOCK_R), jnp.float32),
                        pltpu.VMEM((16, _BLOCK_R), jnp.float32)],
    )(logits_input, labels, jnp.asarray(_UP_COL))
    return out.reshape(1)


# speedup vs baseline: 1.1192x; 1.0055x over previous
import functools

import numpy as np
import jax
import jax.numpy as jnp
from jax.experimental import pallas as pl
from jax.experimental.pallas import tpu as pltpu

_N_BINS = 15
_BLOCK_R = 32768

_bounds = np.linspace(0.0, 1.0, _N_BINS + 1, dtype=np.float32)
_UP_COL = np.full((16, 1), 2.0, dtype=np.float32)
_UP_COL[:14, 0] = _bounds[1:15]
_UP_COL[14, 0] = -1.0


def _ece_block_kernel(x_ref, lab_ref, up_ref, out_ref,
                      cnt_ref, cf_ref, ac_ref, *, inv_n):
    i = pl.program_id(0)

    @pl.when(i == 0)
    def _init():
        cnt_ref[...] = jnp.zeros_like(cnt_ref)
        cf_ref[...] = jnp.zeros_like(cf_ref)
        ac_ref[...] = jnp.zeros_like(ac_ref)

    x = x_ref[...]
    c = x.shape[1]
    m = jnp.max(x, axis=1, keepdims=True)
    ez = jnp.exp(x - m)
    eqb = (x == m).astype(jnp.bfloat16)

    ones_row = jnp.ones((1, c), jnp.float32)
    iota_row = jax.lax.broadcasted_iota(jnp.int32, (1, c), 1).astype(jnp.bfloat16)
    dn = (((1,), (1,)), ((), ()))
    s = jax.lax.dot_general(ones_row, ez, dn,
                            precision=jax.lax.Precision.HIGHEST)
    pred = jax.lax.dot_general(iota_row, eqb, dn,
                               preferred_element_type=jnp.float32)
    conf = 1.0 / s
    lab = lab_ref[0].astype(jnp.float32)
    acc = (pred == lab).astype(jnp.float32)

    mask = (conf > up_ref[...]).astype(jnp.float32)
    cnt_ref[...] += mask
    cf_ref[...] += mask * conf
    ac_ref[...] += mask * acc

    @pl.when(i == pl.num_programs(0) - 1)
    def _fini():
        cum = jnp.concatenate(
            [jnp.sum(cnt_ref[...], axis=1, keepdims=True),
             jnp.sum(cf_ref[...], axis=1, keepdims=True),
             jnp.sum(ac_ref[...], axis=1, keepdims=True)], axis=1)
        total = cum[14:15, :]
        prev = jnp.concatenate([total, cum[0:14, :]], axis=0)
        cur = jnp.concatenate(
            [cum[0:14, :], jnp.zeros((1, 3), jnp.float32)], axis=0)
        stats = prev - cur
        cb = stats[:, 0:1]
        safe = jnp.maximum(cb, 1.0)
        contrib = jnp.abs(stats[:, 1:2] - stats[:, 2:3]) / safe * (cb * inv_n)
        contrib = jnp.where(cb > 0.0, contrib, 0.0)
        out_ref[...] = jnp.sum(contrib, axis=0, keepdims=True)


def kernel(logits_input, labels_input):
    n, c = logits_input.shape
    grid = n // _BLOCK_R
    labels = labels_input.astype(jnp.int32).reshape(grid, 1, _BLOCK_R)
    out = pl.pallas_call(
        functools.partial(_ece_block_kernel, inv_n=1.0 / n),
        grid=(grid,),
        in_specs=[
            pl.BlockSpec((_BLOCK_R, c), lambda i: (i, 0)),
            pl.BlockSpec((1, 1, _BLOCK_R), lambda i: (i, 0, 0)),
            pl.BlockSpec((16, 1), lambda i: (0, 0)),
        ],
        out_specs=pl.BlockSpec((1, 1), lambda i: (0, 0)),
        out_shape=jax.ShapeDtypeStruct((1, 1), jnp.float32),
        scratch_shapes=[pltpu.VMEM((16, _BLOCK_R), jnp.float32),
                        pltpu.VMEM((16, _BLOCK_R), jnp.float32),
                        pltpu.VMEM((16, _BLOCK_R), jnp.float32)],
    )(logits_input, labels, jnp.asarray(_UP_COL))
    return out.reshape(1)


# speedup vs baseline: 1.4850x; 1.3269x over previous
import functools

import numpy as np
import jax
import jax.numpy as jnp
from jax.experimental import pallas as pl
from jax.experimental.pallas import tpu as pltpu

_N_BINS = 15
_BLOCK_R = 16384

_bounds = np.linspace(0.0, 1.0, _N_BINS + 1, dtype=np.float32)
_UP_COL = np.full((16, 1), 2.0, dtype=np.float32)
_UP_COL[:14, 0] = _bounds[1:15]
_UP_COL[14, 0] = -1.0


def _ece_block_kernel(x_ref, lab_ref, up_ref, out_ref,
                      cnt_ref, cf_ref, ac_ref, *, inv_n):
    i = pl.program_id(0)

    @pl.when(i == 0)
    def _init():
        cnt_ref[...] = jnp.zeros_like(cnt_ref)
        cf_ref[...] = jnp.zeros_like(cf_ref)
        ac_ref[...] = jnp.zeros_like(ac_ref)

    x = x_ref[...]
    c = x.shape[1]
    m = jnp.max(x, axis=1, keepdims=True)
    ez = jnp.exp(x - m)
    eqb = (x == m).astype(jnp.bfloat16)

    ez_hi = ez.astype(jnp.bfloat16)
    ez_lo = (ez - ez_hi.astype(jnp.float32)).astype(jnp.bfloat16)
    ones_row = jnp.ones((1, c), jnp.bfloat16)
    iota_row = jax.lax.broadcasted_iota(jnp.int32, (1, c), 1).astype(jnp.bfloat16)
    dn = (((1,), (1,)), ((), ()))
    s_hi = jax.lax.dot_general(ones_row, ez_hi, dn,
                               preferred_element_type=jnp.float32)
    s_lo = jax.lax.dot_general(ones_row, ez_lo, dn,
                               preferred_element_type=jnp.float32)
    s = s_hi + s_lo
    pred = jax.lax.dot_general(iota_row, eqb, dn,
                               preferred_element_type=jnp.float32)
    conf = 1.0 / s
    lab = lab_ref[0].astype(jnp.float32)
    acc = (pred == lab).astype(jnp.float32)

    mask = (conf > up_ref[...]).astype(jnp.float32)
    cnt_ref[...] += mask
    cf_ref[...] += mask * conf
    ac_ref[...] += mask * acc

    @pl.when(i == pl.num_programs(0) - 1)
    def _fini():
        cum = jnp.concatenate(
            [jnp.sum(cnt_ref[...], axis=1, keepdims=True),
             jnp.sum(cf_ref[...], axis=1, keepdims=True),
             jnp.sum(ac_ref[...], axis=1, keepdims=True)], axis=1)
        total = cum[14:15, :]
        prev = jnp.concatenate([total, cum[0:14, :]], axis=0)
        cur = jnp.concatenate(
            [cum[0:14, :], jnp.zeros((1, 3), jnp.float32)], axis=0)
        stats = prev - cur
        cb = stats[:, 0:1]
        safe = jnp.maximum(cb, 1.0)
        contrib = jnp.abs(stats[:, 1:2] - stats[:, 2:3]) / safe * (cb * inv_n)
        contrib = jnp.where(cb > 0.0, contrib, 0.0)
        out_ref[...] = jnp.sum(contrib, axis=0, keepdims=True)


def kernel(logits_input, labels_input):
    n, c = logits_input.shape
    grid = n // _BLOCK_R
    labels = labels_input.astype(jnp.int32).reshape(grid, 1, _BLOCK_R)
    out = pl.pallas_call(
        functools.partial(_ece_block_kernel, inv_n=1.0 / n),
        grid=(grid,),
        in_specs=[
            pl.BlockSpec((_BLOCK_R, c), lambda i: (i, 0)),
            pl.BlockSpec((1, 1, _BLOCK_R), lambda i: (i, 0, 0)),
            pl.BlockSpec((16, 1), lambda i: (0, 0)),
        ],
        out_specs=pl.BlockSpec((1, 1), lambda i: (0, 0)),
        out_shape=jax.ShapeDtypeStruct((1, 1), jnp.float32),
        scratch_shapes=[pltpu.VMEM((16, _BLOCK_R), jnp.float32),
                        pltpu.VMEM((16, _BLOCK_R), jnp.float32),
                        pltpu.VMEM((16, _BLOCK_R), jnp.float32)],
    )(logits_input, labels, jnp.asarray(_UP_COL))
    return out.reshape(1)
